# Initial kernel scaffold; baseline (speedup 1.0000x reference)
import functools

import jax
import jax.numpy as jnp
from jax.experimental import pallas as pl
from jax.experimental.pallas import tpu as pltpu
from jax.experimental.pallas import tpu_sc as plsc

NUM_HEADS = 16
NUM_BUCKETS = 32
QLEN = 2048
KLEN = 2048
NSHIFT = 8
WS = 4096

_THRESHOLDS = (1, 2, 3, 4, 5, 6, 7, 8, 12, 16, 23, 32, 46, 64, 91)


def _fpre_body(table_ref, out_ref):
    tbl = table_ref[...]
    for s in range(NSHIFT):
        l = jax.lax.broadcasted_iota(jnp.int32, (NUM_BUCKETS, WS), 1)
        d = l + (s - (QLEN - 1))
        x = jnp.abs(d)
        g = jnp.zeros((NUM_BUCKETS, WS), jnp.int32)
        for t in _THRESHOLDS:
            g = g + (x >= t).astype(jnp.int32)
        bucket = g + 16 * (d > 0).astype(jnp.int32)
        b_ids = jax.lax.broadcasted_iota(jnp.int32, (NUM_BUCKETS, WS), 0)
        onehot = (bucket == b_ids).astype(jnp.float32)
        f_s = jax.lax.dot_general(
            tbl, onehot, (((0,), (0,)), ((), ())),
            preferred_element_type=jnp.float32,
        )
        out_ref[:, s, :] = f_s


_fpre_call = pl.pallas_call(
    _fpre_body,
    out_shape=jax.ShapeDtypeStruct((NUM_HEADS, NSHIFT, WS), jnp.float32),
)


def _make_expand():
    mesh = plsc.VectorSubcoreMesh(core_axis_name="c", subcore_axis_name="s")

    @functools.partial(
        pl.kernel,
        out_type=jax.ShapeDtypeStruct((NUM_HEADS * QLEN, KLEN), jnp.float32),
        mesh=mesh,
        scratch_types=[
            pltpu.VMEM((NSHIFT * WS,), jnp.float32),
            pltpu.SemaphoreType.DMA,
        ],
    )
    def expand(fpre_hbm, out_hbm, fv, sem):
        c = jax.lax.axis_index("c")
        sid = jax.lax.axis_index("s")
        wid = sid * 2 + c
        h = wid // 2
        for s in range(NSHIFT):
            pltpu.sync_copy(fpre_hbm.at[h, s], fv.at[pl.ds(s * WS, WS)])

        rbase = wid * (QLEN // 2)

        def src_slice(r):
            o = (QLEN - 1) - (r & (QLEN - 1))
            b = o & 7
            off = b * WS + (o - b)
            return fv.at[pl.ds(off, KLEN)]

        def chunk(i, carry):
            for j in range(8):
                r = rbase + i * 8 + j
                pltpu.async_copy(src_slice(r), out_hbm.at[r], sem)
            for j in range(8):
                r = rbase + i * 8 + j
                pltpu.make_async_copy(src_slice(r), out_hbm.at[r], sem).wait()
            return carry

        jax.lax.fori_loop(0, QLEN // 2 // 8, chunk, 0)

    return expand


_expand_call = _make_expand()


@jax.jit
def _impl(table):
    fpre = _fpre_call(table)
    flat = _expand_call(fpre)
    return flat.reshape(1, NUM_HEADS, QLEN, KLEN)


def kernel(query_length, key_length, table):
    return _impl(table)


# speedup vs baseline: 42.6830x; 42.6830x over previous
import functools

import jax
import jax.numpy as jnp
from jax.experimental import pallas as pl
from jax.experimental.pallas import tpu as pltpu
from jax.experimental.pallas import tpu_sc as plsc

NUM_HEADS = 16
NUM_BUCKETS = 32
QLEN = 2048
KLEN = 2048
NSHIFT = 8
WS = 4096

_THRESHOLDS = (1, 2, 3, 4, 5, 6, 7, 8, 12, 16, 23, 32, 46, 64, 91)


def _fpre_body(table_ref, out_ref):
    tbl = table_ref[...]
    for s in range(NSHIFT):
        l = jax.lax.broadcasted_iota(jnp.int32, (NUM_BUCKETS, WS), 1)
        d = l + (s - (QLEN - 1))
        x = jnp.abs(d)
        g = jnp.zeros((NUM_BUCKETS, WS), jnp.int32)
        for t in _THRESHOLDS:
            g = g + (x >= t).astype(jnp.int32)
        bucket = g + 16 * (d > 0).astype(jnp.int32)
        b_ids = jax.lax.broadcasted_iota(jnp.int32, (NUM_BUCKETS, WS), 0)
        onehot = (bucket == b_ids).astype(jnp.float32)
        f_s = jax.lax.dot_general(
            tbl, onehot, (((0,), (0,)), ((), ())),
            preferred_element_type=jnp.float32,
        )
        out_ref[:, s, :] = f_s


_fpre_call = pl.pallas_call(
    _fpre_body,
    out_shape=jax.ShapeDtypeStruct((NUM_HEADS, NSHIFT, WS), jnp.float32),
)


@functools.lru_cache(maxsize=1)
def _make_expand():
    mesh = plsc.VectorSubcoreMesh(
        core_axis_name="c", subcore_axis_name="s", num_cores=2, num_subcores=16
    )

    @functools.partial(
        pl.kernel,
        out_type=jax.ShapeDtypeStruct((NUM_HEADS * QLEN * KLEN,), jnp.float32),
        mesh=mesh,
        scratch_types=[
            pltpu.VMEM((NSHIFT * WS,), jnp.float32),
            pltpu.SemaphoreType.DMA,
        ],
    )
    def expand(fpre_hbm, out_hbm, fv, sem):
        c = jax.lax.axis_index("c")
        sid = jax.lax.axis_index("s")
        wid = sid * 2 + c
        h = wid // 2
        for s in range(NSHIFT):
            pltpu.sync_copy(fpre_hbm.at[h, s], fv.at[pl.ds(s * WS, WS)])

        rbase = wid * (QLEN // 2)

        def src_slice(r):
            o = (QLEN - 1) - (r & (QLEN - 1))
            b = o & 7
            off = pl.multiple_of(b * WS + (o - b), 8)
            return fv.at[pl.ds(off, KLEN)]

        def dst_slice(r):
            return out_hbm.at[pl.ds(pl.multiple_of(r * KLEN, 8), KLEN)]

        def chunk(i, carry):
            for j in range(8):
                r = rbase + i * 8 + j
                pltpu.async_copy(src_slice(r), dst_slice(r), sem)
            for j in range(8):
                r = rbase + i * 8 + j
                pltpu.make_async_copy(src_slice(r), dst_slice(r), sem).wait()
            return carry

        jax.lax.fori_loop(0, QLEN // 2 // 8, chunk, 0)

    return expand


@jax.jit
def _impl(table):
    fpre = _fpre_call(table)
    flat = _make_expand()(fpre)
    return flat.reshape(1, NUM_HEADS, QLEN, KLEN)


def kernel(query_length, key_length, table):
    return _impl(table)
